# baseline (speedup 1.0000x reference)
import dataclasses
import functools

import jax
import jax.numpy as jnp
from jax import lax
from jax.experimental import pallas as pl
from jax.experimental.pallas import tpu as pltpu
from jax.experimental.pallas import tpu_sc as plsc

N_NODES = 10000
N_PAD = 10240
E_EDGES = 160000
DH = 128
INTERP = False


def _silu(v):
    return jax.nn.silu(v)


def _pre0_body(x_ref, nodeW_ref, nodeb_ref, mW1a_ref, mW1b_ref,
               x0_ref, ta_ref, tb_ref):
    x0 = jnp.dot(x_ref[...], nodeW_ref[...],
                 preferred_element_type=jnp.float32, precision=jax.lax.Precision.HIGHEST) + nodeb_ref[...]
    x0_ref[...] = x0
    a = jnp.dot(x0, mW1a_ref[...], preferred_element_type=jnp.float32, precision=jax.lax.Precision.HIGHEST)
    b = jnp.dot(x0, mW1b_ref[...], preferred_element_type=jnp.float32, precision=jax.lax.Precision.HIGHEST)
    ta_ref[0] = a[:, :DH]
    ta_ref[1] = a[:, DH:]
    tb_ref[0] = b[:, :DH]
    tb_ref[1] = b[:, DH:]


def _tc_pre0(x, nodeW, nodeb, mW1a, mW1b):
    full = lambda shp: pl.BlockSpec(shp, lambda i: tuple(0 for _ in shp))
    return pl.pallas_call(
        _pre0_body,
        grid=(_R_GRID,),
        in_specs=[
            pl.BlockSpec((_R_BLK, 128), lambda i: (i, 0)),
            full((128, 256)), full((256,)), full((256, 256)), full((256, 256)),
        ],
        out_specs=[
            pl.BlockSpec((_R_BLK, 256), lambda i: (i, 0)),
            pl.BlockSpec((2, _R_BLK, DH), lambda i: (0, i, 0)),
            pl.BlockSpec((2, _R_BLK, DH), lambda i: (0, i, 0)),
        ],
        out_shape=[
            jax.ShapeDtypeStruct((N_NODES, 256), jnp.float32),
            jax.ShapeDtypeStruct((2, N_PAD, DH), jnp.float32),
            jax.ShapeDtypeStruct((2, N_PAD, DH), jnp.float32),
        ],
        interpret=INTERP,
    )(x, nodeW, nodeb, mW1a, mW1b)


_E_BLK = 2000
_E_GRID = E_EDGES // _E_BLK


def _eterm_body(ea_ref, d2_ref, edgeW_ref, edgeb_ref, WE_ref, WD_ref, MB_ref,
                *out_refs):
    ec = jnp.dot(ea_ref[...], edgeW_ref[...],
                 preferred_element_type=jnp.float32, precision=jax.lax.Precision.HIGHEST) + edgeb_ref[...]
    dist = jnp.sqrt(d2_ref[...] + 1e-12)
    for l in range(4):
        dist_b = dist.astype(jnp.bfloat16).astype(jnp.float32)
        wd_b = WD_ref[pl.ds(l, 1)].astype(jnp.bfloat16).astype(jnp.float32)
        base = (jnp.dot(ec, WE_ref[l], preferred_element_type=jnp.float32, precision=jax.lax.Precision.HIGHEST)
                + dist_b * wd_b + MB_ref[pl.ds(l, 1)])
        out_refs[l][0] = base[:, :DH]
        out_refs[l][1] = base[:, DH:]


def _tc_eterm(edge_attr, d2, edgeW, edgeb, WEs, WDs, MBs, e_pad):
    et_shape = jax.ShapeDtypeStruct((2, e_pad, DH), jnp.float32)
    return pl.pallas_call(
        _eterm_body,
        grid=(_E_GRID,),
        in_specs=[
            pl.BlockSpec((_E_BLK, 16), lambda i: (i, 0)),
            pl.BlockSpec((_E_BLK, 1), lambda i: (i, 0)),
            pl.BlockSpec((16, 16), lambda i: (0, 0)),
            pl.BlockSpec((16,), lambda i: (0,)),
            pl.BlockSpec((4, 16, 256), lambda i: (0, 0, 0)),
            pl.BlockSpec((4, 256), lambda i: (0, 0)),
            pl.BlockSpec((4, 256), lambda i: (0, 0)),
        ],
        out_specs=[pl.BlockSpec((2, _E_BLK, DH), lambda i: (0, i, 0))] * 4,
        out_shape=[et_shape] * 4,
        interpret=INTERP,
    )(edge_attr, d2, edgeW, edgeb, WEs, WDs, MBs)


_R_BLK = 2000
_R_GRID = N_NODES // _R_BLK


def _post_body(has_next, x_ref, s_ref, cnt_ref, mW2_ref, mb2_ref, uW1a_ref,
               uW1b_ref, ub1_ref, uW2_ref, ub2_ref, g_ref, b_ref,
               *next_and_out):
    if has_next:
        mW1a_ref, mW1b_ref, y_ref, ta_ref, tb_ref = next_and_out
    else:
        (y_ref,) = next_and_out
    x = x_ref[...]
    cnt = cnt_ref[0, :, 0:1] + cnt_ref[1, :, 0:1]
    inv = 1.0 / jnp.maximum(cnt, 1.0)
    nz = cnt * inv
    ssum = (jnp.dot(s_ref[0], mW2_ref[pl.ds(0, DH)],
                    preferred_element_type=jnp.float32,
                    precision=jax.lax.Precision.HIGHEST)
            + jnp.dot(s_ref[1], mW2_ref[pl.ds(DH, DH)],
                      preferred_element_type=jnp.float32,
                      precision=jax.lax.Precision.HIGHEST))
    agg = ssum * inv + mb2_ref[...] * nz
    h2 = _silu(jnp.dot(x, uW1a_ref[...], preferred_element_type=jnp.float32, precision=jax.lax.Precision.HIGHEST)
               + jnp.dot(agg, uW1b_ref[...], preferred_element_type=jnp.float32, precision=jax.lax.Precision.HIGHEST)
               + ub1_ref[...])
    upd = jnp.dot(h2, uW2_ref[...], preferred_element_type=jnp.float32, precision=jax.lax.Precision.HIGHEST) + ub2_ref[...]
    r = x + upd
    m = jnp.mean(r, axis=-1, keepdims=True)
    c = r - m
    v = jnp.mean(c * c, axis=-1, keepdims=True)
    y = c / jnp.sqrt(v + 1e-5) * g_ref[...] + b_ref[...]
    y_ref[...] = y
    if has_next:
        a = jnp.dot(y, mW1a_ref[...], preferred_element_type=jnp.float32, precision=jax.lax.Precision.HIGHEST)
        b = jnp.dot(y, mW1b_ref[...], preferred_element_type=jnp.float32, precision=jax.lax.Precision.HIGHEST)
        ta_ref[0] = a[:, :DH]
        ta_ref[1] = a[:, DH:]
        tb_ref[0] = b[:, :DH]
        tb_ref[1] = b[:, DH:]


def _tc_post(x, s, cnt16, p, nxt):
    has_next = nxt is not None
    full = lambda shp: pl.BlockSpec(shp, lambda i: tuple(0 for _ in shp))
    in_specs = [
        pl.BlockSpec((_R_BLK, 256), lambda i: (i, 0)),
        pl.BlockSpec((2, _R_BLK, DH), lambda i: (0, i, 0)),
        pl.BlockSpec((2, _R_BLK, 128), lambda i: (0, i, 0)),
        full((256, 256)), full((256,)), full((256, 256)), full((256, 256)),
        full((256,)), full((256, 256)), full((256,)), full((256,)), full((256,)),
    ]
    out_shape = [jax.ShapeDtypeStruct((N_NODES, 256), jnp.float32)]
    out_specs = [pl.BlockSpec((_R_BLK, 256), lambda i: (i, 0))]
    args = [x, s, cnt16, p['mW2'], p['mb2'], p['uW1'][:256], p['uW1'][256:],
            p['ub1'], p['uW2'], p['ub2'], p['g'], p['b']]
    if has_next:
        in_specs += [full((256, 256)), full((256, 256))]
        args += [nxt['mW1'][:256], nxt['mW1'][256:512]]
        out_shape += [jax.ShapeDtypeStruct((2, N_PAD, DH), jnp.float32)] * 2
        out_specs += [pl.BlockSpec((2, _R_BLK, DH), lambda i: (0, i, 0))] * 2
    return pl.pallas_call(
        functools.partial(_post_body, has_next),
        grid=(_R_GRID,),
        in_specs=in_specs,
        out_specs=out_specs,
        out_shape=out_shape,
        interpret=INTERP,
    )(*args)


def _head_body(x_ref, batch_ref, kvW2_ref, kvb2_ref, oW_ref, ob_ref,
               hW1_ref, hb1_ref, hW2_ref, hb2_ref, logits_ref, w_ref,
               dsum_ref, cnt_ref):
    i = pl.program_id(0)
    x = x_ref[...]
    bcol = batch_ref[:, 0:1]
    iota = jax.lax.broadcasted_iota(jnp.int32, (_R_BLK, 256), 1)
    onehot = (bcol == iota).astype(jnp.float32)
    part = jax.lax.dot_general(
        onehot, x, (((0,), (0,)), ((), ())),
        preferred_element_type=jnp.float32,
        precision=jax.lax.Precision.HIGHEST)
    pcnt = jnp.sum(onehot, axis=0)[:, None]

    @pl.when(i == 0)
    def _():
        dsum_ref[...] = jnp.zeros_like(dsum_ref)
        cnt_ref[...] = jnp.zeros_like(cnt_ref)

    dsum_ref[...] += part
    cnt_ref[...] += jnp.broadcast_to(pcnt, cnt_ref.shape)

    @pl.when(i == _R_GRID - 1)
    def _():
        _head_tail(dsum_ref[...], cnt_ref[:, 0:1], kvW2_ref, kvb2_ref, oW_ref,
                   ob_ref, hW1_ref, hb1_ref, hW2_ref, hb2_ref, logits_ref, w_ref)


def _head_tail(drug_sum, cntb, kvW2_ref, kvb2_ref, oW_ref, ob_ref,
               hW1_ref, hb1_ref, hW2_ref, hb2_ref, logits_ref, w_ref):
    drug = drug_sum * (1.0 / jnp.maximum(cntb, 1.0))
    v = jnp.dot(drug, kvW2_ref[...], preferred_element_type=jnp.float32, precision=jax.lax.Precision.HIGHEST) + kvb2_ref[...]
    combined = jnp.dot(v, oW_ref[...], preferred_element_type=jnp.float32, precision=jax.lax.Precision.HIGHEST) + ob_ref[...]
    hh = jnp.maximum(
        jnp.dot(combined, hW1_ref[...], preferred_element_type=jnp.float32, precision=jax.lax.Precision.HIGHEST)
        + hb1_ref[...], 0.0)
    lg = jnp.dot(hh, hW2_ref[...], preferred_element_type=jnp.float32, precision=jax.lax.Precision.HIGHEST) + hb2_ref[...]
    logits_ref[...] = lg
    w_ref[...] = jnp.ones((256, 8), jnp.float32)


def _tc_head(x, batch16, params):
    full = lambda shp: pl.BlockSpec(shp, lambda i: tuple(0 for _ in shp))
    return pl.pallas_call(
        _head_body,
        grid=(_R_GRID,),
        in_specs=[
            pl.BlockSpec((_R_BLK, 256), lambda i: (i, 0)),
            pl.BlockSpec((_R_BLK, 16), lambda i: (i, 0)),
            full((256, 256)), full((256,)), full((256, 256)), full((256,)),
            full((256, 512)), full((512,)), full((512, 1)), full((1,)),
        ],
        out_specs=[full((256, 1)), full((256, 8))],
        out_shape=[
            jax.ShapeDtypeStruct((256, 1), jnp.float32),
            jax.ShapeDtypeStruct((256, 8), jnp.float32),
        ],
        scratch_shapes=[
            pltpu.VMEM((256, 256), jnp.float32),
            pltpu.VMEM((256, 128), jnp.float32),
        ],
        interpret=INTERP,
    )(x, batch16, params['kvW'][:, 256:], params['kvb'][256:],
      params['oW'], params['ob'], params['mW1'], params['mb1'],
      params['mW2'], params['mb2'])


E_PAD = 163840
_CHUNKS = 80
_E_TILE = E_PAD // 16
_D2_TILE = E_EDGES // 32
_ROWS_TILE = N_PAD // 16
_CNT_CHUNKS = 40


def _sc_mesh():
    return plsc.VectorSubcoreMesh(core_axis_name="c", subcore_axis_name="s")


def _sc_compiler_params():
    cp = pltpu.CompilerParams()
    if "needs_layout_passes" in pltpu.CompilerParams.__dataclass_fields__:
        cp = dataclasses.replace(cp, needs_layout_passes=False)
    return cp


def _sc_d2_body(posx, posy, posz, dstf, srcf, d2_out,
                px, py, pz, dv, sv, d2v):
    c = lax.axis_index("c")
    s = lax.axis_index("s")
    w = c * 16 + s
    base = w * _D2_TILE
    pltpu.sync_copy(posx, px)
    pltpu.sync_copy(posy, py)
    pltpu.sync_copy(posz, pz)
    pltpu.sync_copy(dstf.at[pl.ds(base, _D2_TILE)], dv)
    pltpu.sync_copy(srcf.at[pl.ds(base, _D2_TILE)], sv)

    def step(o):
        vd = dv[pl.ds(o, 16)]
        vs = sv[pl.ds(o, 16)]
        dx = plsc.load_gather(px, [vd]) - plsc.load_gather(px, [vs])
        dy = plsc.load_gather(py, [vd]) - plsc.load_gather(py, [vs])
        dz = plsc.load_gather(pz, [vd]) - plsc.load_gather(pz, [vs])
        d2v[pl.ds(o, 16)] = dx * dx + dy * dy + dz * dz

    @pl.loop(0, _D2_TILE // 16)
    def _(i):
        step(i * 16)

    step(_D2_TILE - 16)
    pltpu.sync_copy(d2v, d2_out.at[pl.ds(base, _D2_TILE)])


def _sc_d2(posx, posy, posz, dst, src):
    k = pl.kernel(
        _sc_d2_body,
        compiler_params=_sc_compiler_params(),
        out_type=jax.ShapeDtypeStruct((E_EDGES,), jnp.float32),
        mesh=_sc_mesh(),
        scratch_types=[
            pltpu.VMEM((N_NODES,), jnp.float32),
            pltpu.VMEM((N_NODES,), jnp.float32),
            pltpu.VMEM((N_NODES,), jnp.float32),
            pltpu.VMEM((_D2_TILE,), jnp.int32),
            pltpu.VMEM((_D2_TILE,), jnp.int32),
            pltpu.VMEM((_D2_TILE,), jnp.float32),
        ],
    )
    return k(posx, posy, posz, dst, src)


def _sc_cnt_body(dstp2, ones_h, zeros_h, cnt_out, idx2, onesv, cnt_sp):
    c = lax.axis_index("c")
    s = lax.axis_index("s")
    row0 = s * _ROWS_TILE
    for r in range(0, _ROWS_TILE, 128):
        pltpu.sync_copy(zeros_h, cnt_sp.at[pl.ds(row0 + r, 128)])
    pltpu.sync_copy(ones_h, onesv)
    pltpu.sync_copy(dstp2.at[pl.ds((c * 16 + s) * _CNT_CHUNKS, _CNT_CHUNKS)], idx2)
    plsc.subcore_barrier()

    @pl.loop(0, _CNT_CHUNKS)
    def _(j):
        pltpu.sync_copy(onesv, cnt_sp.at[idx2.at[j]], add=True)

    plsc.subcore_barrier()
    pltpu.sync_copy(cnt_sp.at[pl.ds(row0, _ROWS_TILE)],
                    cnt_out.at[pl.ds(c * N_PAD + row0, _ROWS_TILE)])


def _sc_cnt(dst_pc, ones_h, zeros_h):
    k = pl.kernel(
        _sc_cnt_body,
        out_type=jax.ShapeDtypeStruct((2 * N_PAD, 128), jnp.float32),
        mesh=_sc_mesh(),
        scratch_types=[
            pltpu.VMEM((_CNT_CHUNKS, 128), jnp.int32),
            pltpu.VMEM((128, 128), jnp.float32),
            pltpu.VMEM_SHARED((N_PAD, 128), jnp.float32),
        ],
    )
    return k(dst_pc, ones_h, zeros_h)


_C_EDGE = 64
_LCHUNKS = _E_TILE // _C_EDGE
_RING = 32


def _sc_layer_body(ta, tb, et, dstp2, srcp2, zeros_h,
                   s_out,
                   idxd, idxs, idxg, rA, rB, rE, s_sp):
    c = lax.axis_index("c")
    s = lax.axis_index("s")
    row0 = s * _ROWS_TILE
    for r in range(0, _ROWS_TILE, 128):
        pltpu.sync_copy(zeros_h, s_sp.at[pl.ds(row0 + r, 128)])
    base = c * E_PAD + s * _E_TILE
    off = jnp.full((16,), c * N_PAD, jnp.int32)
    plsc.subcore_barrier()

    for rr in range(_LCHUNKS // _RING):
        pltpu.sync_copy(dstp2.at[pl.ds(s * _LCHUNKS + rr * _RING, _RING)], idxd)
        pltpu.sync_copy(srcp2.at[pl.ds(s * _LCHUNKS + rr * _RING, _RING)], idxs)

        @pl.loop(0, _RING)
        def _(r):
            for kk in range(_C_EDGE // 16):
                sl = (r, pl.ds(kk * 16, 16))
                idxg[sl] = idxd[sl] + off
                idxs[sl] = idxs[sl] + off

        @pl.loop(0, _RING)
        def _(jj):
            j = rr * _RING + jj
            pltpu.sync_copy(ta.at[idxg.at[jj]], rA)
            pltpu.sync_copy(tb.at[idxs.at[jj]], rB)
            pltpu.sync_copy(et.at[pl.ds(base + j * _C_EDGE, _C_EDGE)], rE)

            @pl.loop(0, _C_EDGE)
            def _(r):
                for kk in range(8):
                    sl = (r, pl.ds(kk * 16, 16))
                    v = rA[sl] + rB[sl] + rE[sl]
                    rE[sl] = v * (1.0 / (1.0 + jnp.exp(-v)))

            pltpu.sync_copy(rE, s_sp.at[idxd.at[jj]], add=True)

    plsc.subcore_barrier()
    pltpu.sync_copy(s_sp.at[pl.ds(row0, _ROWS_TILE)],
                    s_out.at[pl.ds(c * N_PAD + row0, _ROWS_TILE)])


def _sc_layer(ta_flat, tb_flat, et_flat, dst_p, src_p, zeros_h):
    k = pl.kernel(
        _sc_layer_body,
        out_type=jax.ShapeDtypeStruct((2 * N_PAD, DH), jnp.float32),
        mesh=_sc_mesh(),
        scratch_types=[
            pltpu.VMEM((_RING, _C_EDGE), jnp.int32),
            pltpu.VMEM((_RING, _C_EDGE), jnp.int32),
            pltpu.VMEM((_RING, _C_EDGE), jnp.int32),
            pltpu.VMEM((_C_EDGE, DH), jnp.float32),
            pltpu.VMEM((_C_EDGE, DH), jnp.float32),
            pltpu.VMEM((_C_EDGE, DH), jnp.float32),
            pltpu.VMEM_SHARED((N_PAD, DH), jnp.float32),
        ],
    )
    return k(ta_flat, tb_flat, et_flat, dst_p, src_p, zeros_h)


def _sc_layer_jax(ta, tb, et, src, dst):
    h0 = _silu(ta[0][dst] + tb[0][src] + et[0, :E_EDGES])
    h1 = _silu(ta[1][dst] + tb[1][src] + et[1, :E_EDGES])
    s0 = jax.ops.segment_sum(h0, dst, num_segments=N_PAD)
    s1 = jax.ops.segment_sum(h1, dst, num_segments=N_PAD)
    return jnp.stack([s0, s1])


def kernel(x, edge_index, edge_attr, pos, batch, protein_emb, params):
    src = edge_index[0].astype(jnp.int32)
    dst = edge_index[1].astype(jnp.int32)
    layers = params['layers']

    WEs = jnp.stack([p['mW1'][512:528] for p in layers])
    WDs = jnp.stack([p['mW1'][528] for p in layers])
    MBs = jnp.stack([p['mb1'] for p in layers])

    posx = pos[:, 0] + 0.0
    posy = pos[:, 1] + 0.0
    posz = pos[:, 2] + 0.0
    pad = jnp.full((E_PAD - E_EDGES,), N_NODES, jnp.int32)
    dst_pc = jnp.concatenate([dst, pad]).reshape(E_PAD // 128, 128)
    dst_p = jnp.concatenate([dst, pad]).reshape(E_PAD // 64, 64)
    src_p = jnp.concatenate([src, pad]).reshape(E_PAD // 64, 64)
    ones128 = jnp.ones((128, 128), jnp.float32)
    zeros128 = jnp.zeros((128, DH), jnp.float32)

    d2 = _sc_d2(posx, posy, posz, dst, src)
    cnt2 = _sc_cnt(dst_pc, ones128, zeros128).reshape(2, N_PAD, 128)

    x0, ta, tb = _tc_pre0(x, params['nodeW'], params['nodeb'],
                          layers[0]['mW1'][:256], layers[0]['mW1'][256:512])
    ets = _tc_eterm(edge_attr, d2[:, None], params['edgeW'], params['edgeb'],
                    WEs, WDs, MBs, E_PAD)

    xl = x0
    for l in range(4):
        s = _sc_layer_jax(ta, tb, ets[l], src, dst)
        nxt = layers[l + 1] if l < 3 else None
        if nxt is not None:
            xl, ta, tb = _tc_post(xl, s, cnt2, layers[l], nxt)
        else:
            (xl,) = _tc_post(xl, s, cnt2, layers[l], None)

    batch16 = jnp.broadcast_to(batch.astype(jnp.int32)[:, None], (N_NODES, 16))
    logits, w = _tc_head(xl, batch16, params)
    return logits[:, 0], jnp.broadcast_to(w[:, :, None, None], (256, 8, 1, 1))
